# baseline (speedup 1.0000x reference)
import functools

import jax
import jax.numpy as jnp
from jax import lax
from jax.experimental import pallas as pl
from jax.experimental.pallas import tpu as pltpu
from jax.experimental.pallas import tpu_sc as plsc

NC = 2
NS = 16
NW = NC * NS

CHUNK = 128
K = 8

V = 1000000
D = 64
NBLK = V // CHUNK


@jax.jit
def _sc_relayout(wt, wrem_pairs):
    mesh = plsc.VectorSubcoreMesh(
        core_axis_name="c", subcore_axis_name="s", num_cores=NC, num_subcores=NS
    )
    n_iter = NBLK // NW + 1

    @functools.partial(
        pl.kernel,
        out_type=jax.ShapeDtypeStruct((V // 2, 2 * D), jnp.float32),
        mesh=mesh,
        compiler_params=pltpu.CompilerParams(
            use_tc_tiling_on_sc=True, needs_layout_passes=False
        ),
        scratch_types=[
            pltpu.VMEM((D, CHUNK), jnp.float32),
            pltpu.VMEM((D, CHUNK), jnp.float32),
        ],
    )
    def k(wt_hbm, wrem_hbm, tp_hbm, vbuf, tbuf):
        wid = lax.axis_index("s") * NC + lax.axis_index("c")
        lane = lax.iota(jnp.int32, 16)

        def transpose_block(m):
            for j in range(D):
                for half in range(2):
                    col = 2 * j + half
                    for q in range(D // 16):
                        src = plsc.load_gather(
                            vbuf, [lane + q * 16, jnp.full((16,), col, jnp.int32)]
                        )
                        tbuf[j, pl.ds(half * D + q * 16, 16)] = src
            pltpu.sync_copy(tbuf, tp_hbm.at[pl.ds(m * D, D)])

        def body(i, carry):
            m = wid + i * NW

            @pl.when(m < NBLK)
            def _():
                pltpu.sync_copy(wt_hbm.at[:, pl.ds(m * CHUNK, CHUNK)], vbuf)
                transpose_block(m)

            return carry

        lax.fori_loop(0, n_iter, body, 0)

        @pl.when(wid == 0)
        def _():
            pltpu.sync_copy(wrem_hbm, tbuf.at[pl.ds(0, D // 2)])
            pltpu.sync_copy(
                tbuf.at[pl.ds(0, D // 2)], tp_hbm.at[pl.ds(NBLK * D, D // 2)]
            )

    return k(wt, wrem_pairs)


@functools.partial(jax.jit, static_argnums=(2,))
def _sc_gather(idx_flat, table, n_rows):
    per_w = n_rows // NW
    n_chunks = per_w // CHUNK
    n_grp = n_chunks // K
    mesh = plsc.VectorSubcoreMesh(
        core_axis_name="c", subcore_axis_name="s", num_cores=NC, num_subcores=NS
    )

    @functools.partial(
        pl.kernel,
        out_type=jax.ShapeDtypeStruct((n_rows, 2 * D), jnp.float32),
        mesh=mesh,
        compiler_params=pltpu.CompilerParams(use_tc_tiling_on_sc=False),
        scratch_types=[
            pltpu.VMEM((per_w,), jnp.int32),
            pltpu.VMEM((K, CHUNK, D), jnp.float32),
            pltpu.SemaphoreType.DMA((K,)),
        ],
    )
    def k(idx_hbm, table_hbm, out_hbm, idx_v, bufs, gsem):
        wid = lax.axis_index("s") * NC + lax.axis_index("c")
        base = wid * per_w
        pltpu.sync_copy(idx_hbm.at[pl.ds(base, per_w)], idx_v)

        def gather(j, b):
            pltpu.async_copy(
                table_hbm.at[idx_v.at[pl.ds(j * CHUNK, CHUNK)]],
                bufs.at[b],
                gsem.at[b],
            )

        for b in range(K):
            gather(b, b)

        def grp(g, carry):
            for b in range(K):
                j = g * K + b
                pltpu.make_async_copy(
                    table_hbm.at[idx_v.at[pl.ds(0, CHUNK)]], bufs.at[b], gsem.at[b]
                ).wait()
                pltpu.sync_copy(
                    bufs.at[b],
                    out_hbm.at[pl.ds(base + j * CHUNK, CHUNK), pl.ds(0, D)],
                )
                gather(jnp.minimum(j + K, n_chunks - 1), b)
            return carry

        lax.fori_loop(0, n_grp, grp, 0)
        for b in range(K):
            pltpu.make_async_copy(
                table_hbm.at[idx_v.at[pl.ds(0, CHUNK)]], bufs.at[b], gsem.at[b]
            ).wait()

    return k(idx_flat, table)


def kernel(inputs, weight):
    b, s = inputs.shape
    v, d = weight.shape
    idx_flat = inputs.reshape(-1).astype(jnp.int32)
    wrem_pairs = weight[NBLK * CHUNK:].reshape(D // 2, 2 * D)
    table = _sc_relayout(weight.T, wrem_pairs).reshape(v, d)
    out = _sc_gather(idx_flat, table, b * s)
    return out[:, :d].reshape(b, s, d)


# speedup vs baseline: 1.7967x; 1.7967x over previous
import functools

import jax
import jax.numpy as jnp
from jax import lax
from jax.experimental import pallas as pl
from jax.experimental.pallas import tpu as pltpu
from jax.experimental.pallas import tpu_sc as plsc

NC = 2
NS = 16
NW = NC * NS

CHUNK = 128
K = 8

V = 1000000
D = 64
NBLK = V // CHUNK


@jax.jit
def _sc_relayout(wt, wrem_pairs):
    mesh = plsc.VectorSubcoreMesh(
        core_axis_name="c", subcore_axis_name="s", num_cores=NC, num_subcores=NS
    )
    @functools.partial(
        pl.kernel,
        out_type=jax.ShapeDtypeStruct((V // 2, 2 * D), jnp.float32),
        mesh=mesh,
        compiler_params=pltpu.CompilerParams(
            use_tc_tiling_on_sc=True, needs_layout_passes=False
        ),
        scratch_types=[
            pltpu.VMEM((2, D, CHUNK), jnp.float32),
            pltpu.VMEM((2, D, CHUNK), jnp.float32),
            pltpu.SemaphoreType.DMA((2,)),
            pltpu.SemaphoreType.DMA((2,)),
        ],
    )
    def k(wt_hbm, wrem_hbm, tp_hbm, vbufs, tbufs, gsem, wsem):
        wid = lax.axis_index("s") * NC + lax.axis_index("c")
        lane = lax.iota(jnp.int32, 16)
        p_row = lax.shift_right_logical(lane, 1)
        p_col = lax.shift_left(lane & 1, 6)
        n_i = jnp.int32(NBLK // NW) + (wid < NBLK % NW).astype(jnp.int32)

        pltpu.async_copy(
            wt_hbm.at[:, pl.ds(wid * CHUNK, CHUNK)], vbufs.at[0], gsem.at[0]
        )

        def transpose_block(vbuf, tbuf):
            @plsc.parallel_loop(0, (CHUNK // 16) * D, unroll=8)
            def _(t):
                q = lax.shift_right_logical(t, 6)
                c = t & (D - 1)
                vec = vbuf[c, pl.ds(q * 16, 16)]
                plsc.store_scatter(
                    tbuf, [p_row + lax.shift_left(q, 3), p_col + c], vec
                )

        def step(i, b):
            m = wid + i * NW

            @pl.when(i + 1 < n_i)
            def _():
                pltpu.async_copy(
                    wt_hbm.at[:, pl.ds((m + NW) * CHUNK, CHUNK)],
                    vbufs.at[1 - b],
                    gsem.at[1 - b],
                )

            pltpu.make_async_copy(
                wt_hbm.at[:, pl.ds(0, CHUNK)], vbufs.at[b], gsem.at[b]
            ).wait()

            @pl.when(i >= 2)
            def _():
                pltpu.make_async_copy(
                    tbufs.at[b], tp_hbm.at[pl.ds(0, D)], wsem.at[b]
                ).wait()

            transpose_block(vbufs.at[b], tbufs.at[b])
            pltpu.async_copy(tbufs.at[b], tp_hbm.at[pl.ds(m * D, D)], wsem.at[b])

        def grp(g, carry):
            step(2 * g, 0)
            step(2 * g + 1, 1)
            return carry

        lax.fori_loop(0, jnp.int32(NBLK // NW // 2), grp, 0)

        @pl.when(wid < NBLK % NW)
        def _():
            step(jnp.int32(NBLK // NW), 0)

        for b in range(2):
            pltpu.make_async_copy(
                tbufs.at[b], tp_hbm.at[pl.ds(0, D)], wsem.at[b]
            ).wait()

        @pl.when(wid == 0)
        def _():
            pltpu.sync_copy(wrem_hbm, tbufs.at[0].at[pl.ds(0, D // 2)])
            pltpu.sync_copy(
                tbufs.at[0].at[pl.ds(0, D // 2)], tp_hbm.at[pl.ds(NBLK * D, D // 2)]
            )

    return k(wt, wrem_pairs)


@functools.partial(jax.jit, static_argnums=(2,))
def _sc_gather(idx_flat, table, n_rows):
    per_w = n_rows // NW
    n_chunks = per_w // CHUNK
    n_grp = n_chunks // K
    mesh = plsc.VectorSubcoreMesh(
        core_axis_name="c", subcore_axis_name="s", num_cores=NC, num_subcores=NS
    )

    @functools.partial(
        pl.kernel,
        out_type=jax.ShapeDtypeStruct((n_rows, 2 * D), jnp.float32),
        mesh=mesh,
        compiler_params=pltpu.CompilerParams(use_tc_tiling_on_sc=False),
        scratch_types=[
            pltpu.VMEM((per_w,), jnp.int32),
            pltpu.VMEM((K, CHUNK, D), jnp.float32),
            pltpu.SemaphoreType.DMA((K,)),
        ],
    )
    def k(idx_hbm, table_hbm, out_hbm, idx_v, bufs, gsem):
        wid = lax.axis_index("s") * NC + lax.axis_index("c")
        base = wid * per_w
        pltpu.sync_copy(idx_hbm.at[pl.ds(base, per_w)], idx_v)

        def gather(j, b):
            pltpu.async_copy(
                table_hbm.at[idx_v.at[pl.ds(j * CHUNK, CHUNK)]],
                bufs.at[b],
                gsem.at[b],
            )

        for b in range(K):
            gather(b, b)

        def grp(g, carry):
            for b in range(K):
                j = g * K + b
                pltpu.make_async_copy(
                    table_hbm.at[idx_v.at[pl.ds(0, CHUNK)]], bufs.at[b], gsem.at[b]
                ).wait()
                pltpu.sync_copy(
                    bufs.at[b],
                    out_hbm.at[pl.ds(base + j * CHUNK, CHUNK), pl.ds(0, D)],
                )
                gather(jnp.minimum(j + K, n_chunks - 1), b)
            return carry

        lax.fori_loop(0, n_grp, grp, 0)
        for b in range(K):
            pltpu.make_async_copy(
                table_hbm.at[idx_v.at[pl.ds(0, CHUNK)]], bufs.at[b], gsem.at[b]
            ).wait()

    return k(idx_flat, table)


def kernel(inputs, weight):
    b, s = inputs.shape
    v, d = weight.shape
    idx_flat = inputs.reshape(-1).astype(jnp.int32)
    wrem_pairs = weight[NBLK * CHUNK:].reshape(D // 2, 2 * D)
    table = _sc_relayout(weight.T, wrem_pairs).reshape(v, d)
    out = _sc_gather(idx_flat, table, b * s)
    return out[:, :d].reshape(b, s, d)


# speedup vs baseline: 3.5279x; 1.9635x over previous
import functools

import jax
import jax.numpy as jnp
from jax import lax
from jax.experimental import pallas as pl
from jax.experimental.pallas import tpu as pltpu
from jax.experimental.pallas import tpu_sc as plsc

NC = 2
NS = 16
NW = NC * NS

CHUNK = 128
K = 8

V = 1000000
D = 64
NBLK = V // CHUNK


@jax.jit
def _sc_relayout(wt, wrem_pairs):
    mesh = plsc.VectorSubcoreMesh(
        core_axis_name="c", subcore_axis_name="s", num_cores=NC, num_subcores=NS
    )
    @functools.partial(
        pl.kernel,
        out_type=jax.ShapeDtypeStruct((V // 2, 2 * D), jnp.float32),
        mesh=mesh,
        compiler_params=pltpu.CompilerParams(
            use_tc_tiling_on_sc=True, needs_layout_passes=False
        ),
        scratch_types=[
            pltpu.VMEM((2, D, CHUNK), jnp.float32),
            pltpu.VMEM((2, D, CHUNK), jnp.float32),
            pltpu.VMEM((D * (CHUNK + 1),), jnp.float32),
            pltpu.SemaphoreType.DMA((2,)),
            pltpu.SemaphoreType.DMA((2,)),
        ],
    )
    def k(wt_hbm, wrem_hbm, tp_hbm, vbufs, tbufs, sbuf, gsem, wsem):
        wid = lax.axis_index("s") * NC + lax.axis_index("c")
        lane = lax.iota(jnp.int32, 16)
        SKEW = CHUNK + 1
        lane_skew = lane * SKEW
        n_i = jnp.int32(NBLK // NW) + (wid < NBLK % NW).astype(jnp.int32)

        pltpu.async_copy(
            wt_hbm.at[:, pl.ds(wid * CHUNK, CHUNK)], vbufs.at[0], gsem.at[0]
        )

        def transpose_block(vbuf, tbuf):
            @plsc.parallel_loop(0, (CHUNK // 16) * D, unroll=8)
            def _(t):
                q = lax.shift_right_logical(t, 6)
                c = t & (D - 1)
                vec = vbuf[c, pl.ds(q * 16, 16)]
                plsc.store_scatter(sbuf, [lane + (c * SKEW + q * 16)], vec)

            @plsc.parallel_loop(0, 2 * (D // 16) * D, unroll=8)
            def _(t):
                j = lax.shift_right_logical(t, 3)
                h = (t >> 2) & 1
                c0 = lax.shift_left(t & 3, 4)
                vec = plsc.load_gather(
                    sbuf, [lane_skew + (c0 * SKEW + 2 * j + h)]
                )
                tbuf[j, pl.ds(h * D + c0, 16)] = vec

        def step(i, b):
            m = wid + i * NW

            @pl.when(i + 1 < n_i)
            def _():
                pltpu.async_copy(
                    wt_hbm.at[:, pl.ds((m + NW) * CHUNK, CHUNK)],
                    vbufs.at[1 - b],
                    gsem.at[1 - b],
                )

            pltpu.make_async_copy(
                wt_hbm.at[:, pl.ds(0, CHUNK)], vbufs.at[b], gsem.at[b]
            ).wait()

            @pl.when(i >= 2)
            def _():
                pltpu.make_async_copy(
                    tbufs.at[b], tp_hbm.at[pl.ds(0, D)], wsem.at[b]
                ).wait()

            transpose_block(vbufs.at[b], tbufs.at[b])
            pltpu.async_copy(tbufs.at[b], tp_hbm.at[pl.ds(m * D, D)], wsem.at[b])

        def grp(g, carry):
            step(2 * g, 0)
            step(2 * g + 1, 1)
            return carry

        lax.fori_loop(0, jnp.int32(NBLK // NW // 2), grp, 0)

        @pl.when(wid < NBLK % NW)
        def _():
            step(jnp.int32(NBLK // NW), 0)

        for b in range(2):
            pltpu.make_async_copy(
                tbufs.at[b], tp_hbm.at[pl.ds(0, D)], wsem.at[b]
            ).wait()

        @pl.when(wid == 0)
        def _():
            pltpu.sync_copy(wrem_hbm, tbufs.at[0].at[pl.ds(0, D // 2)])
            pltpu.sync_copy(
                tbufs.at[0].at[pl.ds(0, D // 2)], tp_hbm.at[pl.ds(NBLK * D, D // 2)]
            )

    return k(wt, wrem_pairs)


@functools.partial(jax.jit, static_argnums=(2,))
def _sc_gather(idx_flat, table, n_rows):
    per_w = n_rows // NW
    n_chunks = per_w // CHUNK
    n_grp = n_chunks // K
    mesh = plsc.VectorSubcoreMesh(
        core_axis_name="c", subcore_axis_name="s", num_cores=NC, num_subcores=NS
    )

    @functools.partial(
        pl.kernel,
        out_type=jax.ShapeDtypeStruct((n_rows, 2 * D), jnp.float32),
        mesh=mesh,
        compiler_params=pltpu.CompilerParams(use_tc_tiling_on_sc=False),
        scratch_types=[
            pltpu.VMEM((per_w,), jnp.int32),
            pltpu.VMEM((K, CHUNK, D), jnp.float32),
            pltpu.SemaphoreType.DMA((K,)),
        ],
    )
    def k(idx_hbm, table_hbm, out_hbm, idx_v, bufs, gsem):
        wid = lax.axis_index("s") * NC + lax.axis_index("c")
        base = wid * per_w
        pltpu.sync_copy(idx_hbm.at[pl.ds(base, per_w)], idx_v)

        def gather(j, b):
            pltpu.async_copy(
                table_hbm.at[idx_v.at[pl.ds(j * CHUNK, CHUNK)]],
                bufs.at[b],
                gsem.at[b],
            )

        for b in range(K):
            gather(b, b)

        def grp(g, carry):
            for b in range(K):
                j = g * K + b
                pltpu.make_async_copy(
                    table_hbm.at[idx_v.at[pl.ds(0, CHUNK)]], bufs.at[b], gsem.at[b]
                ).wait()
                pltpu.sync_copy(
                    bufs.at[b],
                    out_hbm.at[pl.ds(base + j * CHUNK, CHUNK), pl.ds(0, D)],
                )
                gather(jnp.minimum(j + K, n_chunks - 1), b)
            return carry

        lax.fori_loop(0, n_grp, grp, 0)
        for b in range(K):
            pltpu.make_async_copy(
                table_hbm.at[idx_v.at[pl.ds(0, CHUNK)]], bufs.at[b], gsem.at[b]
            ).wait()

    return k(idx_flat, table)


def kernel(inputs, weight):
    b, s = inputs.shape
    v, d = weight.shape
    idx_flat = inputs.reshape(-1).astype(jnp.int32)
    wrem_pairs = weight[NBLK * CHUNK:].reshape(D // 2, 2 * D)
    table = _sc_relayout(weight.T, wrem_pairs).reshape(v, d)
    out = _sc_gather(idx_flat, table, b * s)
    return out[:, :d].reshape(b, s, d)


# speedup vs baseline: 3.5901x; 1.0176x over previous
import functools

import jax
import jax.numpy as jnp
from jax import lax
from jax.experimental import pallas as pl
from jax.experimental.pallas import tpu as pltpu
from jax.experimental.pallas import tpu_sc as plsc

NC = 2
NS = 16
NW = NC * NS

CHUNK = 128
K = 8

V = 1000000
D = 64
NBLK = V // CHUNK


@jax.jit
def _sc_relayout(wt, wrem_pairs):
    mesh = plsc.VectorSubcoreMesh(
        core_axis_name="c", subcore_axis_name="s", num_cores=NC, num_subcores=NS
    )
    @functools.partial(
        pl.kernel,
        out_type=jax.ShapeDtypeStruct((V // 2, 2 * D), jnp.float32),
        mesh=mesh,
        compiler_params=pltpu.CompilerParams(
            use_tc_tiling_on_sc=True, needs_layout_passes=False
        ),
        scratch_types=[
            pltpu.VMEM((2, D, CHUNK), jnp.float32),
            pltpu.VMEM((2, D, CHUNK), jnp.float32),
            pltpu.VMEM((D * (CHUNK + 1),), jnp.float32),
            pltpu.SemaphoreType.DMA((2,)),
            pltpu.SemaphoreType.DMA((2,)),
        ],
    )
    def k(wt_hbm, wrem_hbm, tp_hbm, vbufs, tbufs, sbuf, gsem, wsem):
        wid = lax.axis_index("s") * NC + lax.axis_index("c")
        lane = lax.iota(jnp.int32, 16)
        SKEW = CHUNK + 1
        lane_skew = lane * SKEW
        n_i = jnp.int32(NBLK // NW) + (wid < NBLK % NW).astype(jnp.int32)

        pltpu.async_copy(
            wt_hbm.at[:, pl.ds(wid * CHUNK, CHUNK)], vbufs.at[0], gsem.at[0]
        )

        def transpose_block(vbuf, tbuf):
            @plsc.parallel_loop(0, (CHUNK // 16) * D, unroll=16)
            def _(t):
                q = lax.shift_right_logical(t, 6)
                c = t & (D - 1)
                vec = vbuf[c, pl.ds(q * 16, 16)]
                plsc.store_scatter(sbuf, [lane + (c * SKEW + q * 16)], vec)

            @plsc.parallel_loop(0, 2 * (D // 16) * D, unroll=16)
            def _(t):
                j = lax.shift_right_logical(t, 3)
                h = (t >> 2) & 1
                c0 = lax.shift_left(t & 3, 4)
                vec = plsc.load_gather(
                    sbuf, [lane_skew + (c0 * SKEW + 2 * j + h)]
                )
                tbuf[j, pl.ds(h * D + c0, 16)] = vec

        def step(i, b):
            m = wid + i * NW

            @pl.when(i + 1 < n_i)
            def _():
                pltpu.async_copy(
                    wt_hbm.at[:, pl.ds((m + NW) * CHUNK, CHUNK)],
                    vbufs.at[1 - b],
                    gsem.at[1 - b],
                )

            pltpu.make_async_copy(
                wt_hbm.at[:, pl.ds(0, CHUNK)], vbufs.at[b], gsem.at[b]
            ).wait()

            @pl.when(i >= 2)
            def _():
                pltpu.make_async_copy(
                    tbufs.at[b], tp_hbm.at[pl.ds(0, D)], wsem.at[b]
                ).wait()

            transpose_block(vbufs.at[b], tbufs.at[b])
            pltpu.async_copy(tbufs.at[b], tp_hbm.at[pl.ds(m * D, D)], wsem.at[b])

        def grp(g, carry):
            step(2 * g, 0)
            step(2 * g + 1, 1)
            return carry

        lax.fori_loop(0, jnp.int32(NBLK // NW // 2), grp, 0)

        @pl.when(wid < NBLK % NW)
        def _():
            step(jnp.int32(NBLK // NW), 0)

        for b in range(2):
            pltpu.make_async_copy(
                tbufs.at[b], tp_hbm.at[pl.ds(0, D)], wsem.at[b]
            ).wait()

        @pl.when(wid == 0)
        def _():
            pltpu.sync_copy(wrem_hbm, tbufs.at[0].at[pl.ds(0, D // 2)])
            pltpu.sync_copy(
                tbufs.at[0].at[pl.ds(0, D // 2)], tp_hbm.at[pl.ds(NBLK * D, D // 2)]
            )

    return k(wt, wrem_pairs)


@functools.partial(jax.jit, static_argnums=(2,))
def _sc_gather(idx_flat, table, n_rows):
    per_w = n_rows // NW
    n_chunks = per_w // CHUNK
    n_grp = n_chunks // K
    mesh = plsc.VectorSubcoreMesh(
        core_axis_name="c", subcore_axis_name="s", num_cores=NC, num_subcores=NS
    )

    @functools.partial(
        pl.kernel,
        out_type=jax.ShapeDtypeStruct((n_rows, 2 * D), jnp.float32),
        mesh=mesh,
        compiler_params=pltpu.CompilerParams(use_tc_tiling_on_sc=False),
        scratch_types=[
            pltpu.VMEM((per_w,), jnp.int32),
            pltpu.VMEM((K, CHUNK, D), jnp.float32),
            pltpu.SemaphoreType.DMA((K,)),
        ],
    )
    def k(idx_hbm, table_hbm, out_hbm, idx_v, bufs, gsem):
        wid = lax.axis_index("s") * NC + lax.axis_index("c")
        base = wid * per_w
        pltpu.sync_copy(idx_hbm.at[pl.ds(base, per_w)], idx_v)

        def gather(j, b):
            pltpu.async_copy(
                table_hbm.at[idx_v.at[pl.ds(j * CHUNK, CHUNK)]],
                bufs.at[b],
                gsem.at[b],
            )

        for b in range(K):
            gather(b, b)

        def grp(g, carry):
            for b in range(K):
                j = g * K + b
                pltpu.make_async_copy(
                    table_hbm.at[idx_v.at[pl.ds(0, CHUNK)]], bufs.at[b], gsem.at[b]
                ).wait()
                pltpu.sync_copy(
                    bufs.at[b],
                    out_hbm.at[pl.ds(base + j * CHUNK, CHUNK), pl.ds(0, D)],
                )
                gather(jnp.minimum(j + K, n_chunks - 1), b)
            return carry

        lax.fori_loop(0, n_grp, grp, 0)
        for b in range(K):
            pltpu.make_async_copy(
                table_hbm.at[idx_v.at[pl.ds(0, CHUNK)]], bufs.at[b], gsem.at[b]
            ).wait()

    return k(idx_flat, table)


def kernel(inputs, weight):
    b, s = inputs.shape
    v, d = weight.shape
    idx_flat = inputs.reshape(-1).astype(jnp.int32)
    wrem_pairs = weight[NBLK * CHUNK:].reshape(D // 2, 2 * D)
    table = _sc_relayout(weight.T, wrem_pairs).reshape(v, d)
    out = _sc_gather(idx_flat, table, b * s)
    return out[:, :d].reshape(b, s, d)
